# baseline (speedup 1.0000x reference)
import jax
import jax.numpy as jnp
from jax import lax
from jax.experimental import pallas as pl
from jax.experimental.pallas import tpu as pltpu
from jax.experimental.pallas import tpu_sc as plsc

_N = 10000
_E = 320000
_NSUB = 16
_NCORE = 2
_NW = _NSUB * _NCORE
_CHUNK = 128
_CHUNKS = 80
_EPAD = _NW * _CHUNKS * _CHUNK
_NACC = 10112
_RPT = _NACC // _NSUB


def _make_edge_scatter(width):
  mesh = plsc.VectorSubcoreMesh(core_axis_name="c", subcore_axis_name="s")

  def body(table_hbm, src_hbm, dst_hbm, zeros_hbm, out_hbm,
           src_v, dst_v, rows0, rows1, acc_sh, sg0, sg1, ss0, ss1):
    c = lax.axis_index("c")
    s = lax.axis_index("s")
    wid = c * _NSUB + s
    pltpu.sync_copy(src_hbm.at[wid], src_v)
    pltpu.sync_copy(dst_hbm.at[wid], dst_v)
    sl = pl.ds(s * _RPT, _RPT)
    pltpu.sync_copy(zeros_hbm, acc_sh.at[sl])
    plsc.subcore_barrier()

    def gather(j, buf, sem):
      pltpu.async_copy(table_hbm.at[src_v.at[j]], buf, sem)

    def gather_wait(buf, sem):
      pltpu.make_async_copy(table_hbm.at[src_v.at[0]], buf, sem).wait()

    def scatter(j, buf, sem):
      pltpu.async_copy(buf, acc_sh.at[dst_v.at[j]], sem, add=True)

    def scatter_wait(buf, sem):
      pltpu.make_async_copy(buf, acc_sh.at[dst_v.at[0]], sem).wait()

    gather(0, rows0, sg0)

    def pair(t, carry):
      j0 = 2 * t
      j1 = j0 + 1
      gather_wait(rows0, sg0)
      scatter(j0, rows0, ss0)

      @pl.when(t >= 1)
      def _():
        scatter_wait(rows1, ss1)

      gather(j1, rows1, sg1)
      gather_wait(rows1, sg1)
      scatter(j1, rows1, ss1)
      scatter_wait(rows0, ss0)

      @pl.when(t < _CHUNKS // 2 - 1)
      def _():
        gather(j0 + 2, rows0, sg0)

      return carry

    lax.fori_loop(0, _CHUNKS // 2, pair, 0)
    scatter_wait(rows1, ss1)
    plsc.subcore_barrier()
    pltpu.sync_copy(acc_sh.at[sl], out_hbm.at[c, sl])

  return pl.kernel(
      body,
      out_type=jax.ShapeDtypeStruct((_NCORE, _NACC, width), jnp.float32),
      mesh=mesh,
      compiler_params=pltpu.CompilerParams(use_tc_tiling_on_sc=False),
      scratch_types=[
          pltpu.VMEM((_CHUNKS, _CHUNK), jnp.int32),
          pltpu.VMEM((_CHUNKS, _CHUNK), jnp.int32),
          pltpu.VMEM((_CHUNK, width), jnp.float32),
          pltpu.VMEM((_CHUNK, width), jnp.float32),
          pltpu.VMEM_SHARED((_NACC, width), jnp.float32),
          pltpu.SemaphoreType.DMA,
          pltpu.SemaphoreType.DMA,
          pltpu.SemaphoreType.DMA,
          pltpu.SemaphoreType.DMA,
      ],
  )


_DEGW = 8


def _make_degree():
  mesh = plsc.VectorSubcoreMesh(core_axis_name="c", subcore_axis_name="s")

  def body(ones_hbm, dst_hbm, zeros_hbm, out_hbm, dst_v, rows_v, acc_sh):
    c = lax.axis_index("c")
    s = lax.axis_index("s")
    wid = c * _NSUB + s
    pltpu.sync_copy(dst_hbm.at[wid], dst_v)
    pltpu.sync_copy(ones_hbm, rows_v)
    sl = pl.ds(s * _RPT, _RPT)
    pltpu.sync_copy(zeros_hbm, acc_sh.at[sl])
    plsc.subcore_barrier()

    def step(j, carry):
      pltpu.sync_copy(rows_v, acc_sh.at[dst_v.at[j]], add=True)
      return carry

    lax.fori_loop(0, _CHUNKS, step, 0)
    plsc.subcore_barrier()
    pltpu.sync_copy(acc_sh.at[sl], out_hbm.at[c, sl])

  return pl.kernel(
      body,
      out_type=jax.ShapeDtypeStruct((_NCORE, _NACC, _DEGW), jnp.float32),
      mesh=mesh,
      compiler_params=pltpu.CompilerParams(use_tc_tiling_on_sc=False),
      scratch_types=[
          pltpu.VMEM((_CHUNKS, _CHUNK), jnp.int32),
          pltpu.VMEM((_CHUNK, _DEGW), jnp.float32),
          pltpu.VMEM_SHARED((_NACC, _DEGW), jnp.float32),
      ],
  )


def _dinv_of(degp_ref):
  deg = degp_ref[0, 0:_N, 0:1] + degp_ref[1, 0:_N, 0:1] + 1.0
  return lax.rsqrt(deg)


def _tc_first_body(x_ref, glove_ref, w1_ref, degp_ref, g1_ref):
  dinv = _dinv_of(degp_ref)
  w1p = jnp.dot(glove_ref[...], w1_ref[...], preferred_element_type=jnp.float32)
  g1_ref[...] = dinv * jnp.dot(x_ref[...], w1p,
                               preferred_element_type=jnp.float32)


def _tc_mid_body(sp_ref, g_ref, degp_ref, b_ref, w_ref, out_ref):
  dinv = _dinv_of(degp_ref)
  ssum = sp_ref[0, 0:_N, :] + sp_ref[1, 0:_N, :]
  h = jnp.maximum(dinv * (ssum + g_ref[...]) + b_ref[...], 0.0)
  out_ref[...] = dinv * jnp.dot(h, w_ref[...],
                                preferred_element_type=jnp.float32)


def _tc_final_body(sp_ref, g_ref, degp_ref, b_ref, out_ref):
  dinv = _dinv_of(degp_ref)
  ssum = sp_ref[0, 0:_N, :] + sp_ref[1, 0:_N, :]
  o = dinv * (ssum + g_ref[...]) + b_ref[...]
  m = jnp.max(o, axis=1, keepdims=True)
  lse = m + jnp.log(jnp.sum(jnp.exp(o - m), axis=1, keepdims=True))
  out_ref[...] = o - lse


_degree = _make_degree()
_scatter32 = _make_edge_scatter(32)
_scatter16 = _make_edge_scatter(16)

_tc_first = pl.pallas_call(
    _tc_first_body,
    out_shape=jax.ShapeDtypeStruct((_N, 32), jnp.float32))


def _tc_mid(width):
  return pl.pallas_call(
      _tc_mid_body,
      out_shape=jax.ShapeDtypeStruct((_N, width), jnp.float32))


_tc_final = pl.pallas_call(
    _tc_final_body,
    out_shape=jax.ShapeDtypeStruct((_N, 16), jnp.float32))


def kernel(x, edge_index, glove, W1, b1, W2, b2, W3, b3):
  pad = _EPAD - _E
  srcp = jnp.concatenate(
      [edge_index[0], jnp.zeros((pad,), jnp.int32)]).reshape(
          _NW, _CHUNKS, _CHUNK)
  dstp = jnp.concatenate(
      [edge_index[1], jnp.full((pad,), _N, jnp.int32)]).reshape(
          _NW, _CHUNKS, _CHUNK)
  ones = jnp.ones((_CHUNK, _DEGW), jnp.float32)
  z8 = jnp.zeros((_RPT, _DEGW), jnp.float32)
  z32 = jnp.zeros((_RPT, 32), jnp.float32)
  z16 = jnp.zeros((_RPT, 16), jnp.float32)

  degp = _degree(ones, dstp, z8)
  g1 = _tc_first(x, glove, W1, degp)
  s1 = _scatter32(g1, srcp, dstp, z32)
  g2 = _tc_mid(32)(s1, g1, degp, b1.reshape(1, -1), W2)
  s2 = _scatter32(g2, srcp, dstp, z32)
  g3 = _tc_mid(16)(s2, g2, degp, b2.reshape(1, -1), W3)
  s3 = _scatter16(g3, srcp, dstp, z16)
  return _tc_final(s3, g3, degp, b3.reshape(1, -1))


# speedup vs baseline: 1.1435x; 1.1435x over previous
import jax
import jax.numpy as jnp
from jax import lax
from jax.experimental import pallas as pl
from jax.experimental.pallas import tpu as pltpu
from jax.experimental.pallas import tpu_sc as plsc

_N = 10000
_E = 320000
_NSUB = 16
_NCORE = 2
_NW = _NSUB * _NCORE
_CHUNK = 128
_CHUNKS = 80
_BLKC = 8
_NBLK = _CHUNKS // _BLKC
_EPAD = _NW * _CHUNKS * _CHUNK
_NACC = 10112
_RPT = _NACC // _NSUB


def _make_edge_scatter(width):
  mesh = plsc.VectorSubcoreMesh(core_axis_name="c", subcore_axis_name="s")

  def body(table_hbm, src_hbm, dst_hbm, zeros_hbm, out_hbm,
           src_v, dst_v, rows_v, acc_sh, sem):
    c = lax.axis_index("c")
    s = lax.axis_index("s")
    wid = c * _NSUB + s
    pltpu.sync_copy(src_hbm.at[wid], src_v)
    pltpu.sync_copy(dst_hbm.at[wid], dst_v)
    sl = pl.ds(s * _RPT, _RPT)
    pltpu.sync_copy(zeros_hbm, acc_sh.at[sl])
    plsc.subcore_barrier()

    def step(j, carry):
      pltpu.async_copy(table_hbm.at[src_v.at[j]], rows_v, sem).wait()
      pltpu.sync_copy(rows_v, acc_sh.at[dst_v.at[j]], add=True)
      return carry

    lax.fori_loop(0, _NBLK, step, 0)
    plsc.subcore_barrier()
    pltpu.sync_copy(acc_sh.at[sl], out_hbm.at[c, sl])

  return pl.kernel(
      body,
      out_type=jax.ShapeDtypeStruct((_NCORE, _NACC, width), jnp.float32),
      mesh=mesh,
      compiler_params=pltpu.CompilerParams(use_tc_tiling_on_sc=False),
      scratch_types=[
          pltpu.VMEM((_NBLK, _BLKC * _CHUNK), jnp.int32),
          pltpu.VMEM((_NBLK, _BLKC * _CHUNK), jnp.int32),
          pltpu.VMEM((_BLKC * _CHUNK, width), jnp.float32),
          pltpu.VMEM_SHARED((_NACC, width), jnp.float32),
          pltpu.SemaphoreType.DMA,
      ],
  )


_DEGW = 8


def _make_degree():
  mesh = plsc.VectorSubcoreMesh(core_axis_name="c", subcore_axis_name="s")

  def body(ones_hbm, dst_hbm, zeros_hbm, out_hbm, dst_v, rows_v, acc_sh):
    c = lax.axis_index("c")
    s = lax.axis_index("s")
    wid = c * _NSUB + s
    pltpu.sync_copy(dst_hbm.at[wid], dst_v)
    pltpu.sync_copy(ones_hbm, rows_v)
    sl = pl.ds(s * _RPT, _RPT)
    pltpu.sync_copy(zeros_hbm, acc_sh.at[sl])
    plsc.subcore_barrier()

    def step(j, carry):
      pltpu.sync_copy(rows_v, acc_sh.at[dst_v.at[j]], add=True)
      return carry

    lax.fori_loop(0, _NBLK, step, 0)
    plsc.subcore_barrier()
    pltpu.sync_copy(acc_sh.at[sl], out_hbm.at[c, sl])

  return pl.kernel(
      body,
      out_type=jax.ShapeDtypeStruct((_NCORE, _NACC, _DEGW), jnp.float32),
      mesh=mesh,
      compiler_params=pltpu.CompilerParams(use_tc_tiling_on_sc=False),
      scratch_types=[
          pltpu.VMEM((_NBLK, _BLKC * _CHUNK), jnp.int32),
          pltpu.VMEM((_BLKC * _CHUNK, _DEGW), jnp.float32),
          pltpu.VMEM_SHARED((_NACC, _DEGW), jnp.float32),
      ],
  )


def _dinv_of(degp_ref):
  deg = degp_ref[0, 0:_N, 0:1] + degp_ref[1, 0:_N, 0:1] + 1.0
  return lax.rsqrt(deg)


def _tc_first_body(x_ref, glove_ref, w1_ref, degp_ref, g1_ref):
  dinv = _dinv_of(degp_ref)
  w1p = jnp.dot(glove_ref[...], w1_ref[...], preferred_element_type=jnp.float32)
  g1_ref[...] = dinv * jnp.dot(x_ref[...], w1p,
                               preferred_element_type=jnp.float32)


def _tc_mid_body(sp_ref, g_ref, degp_ref, b_ref, w_ref, out_ref):
  dinv = _dinv_of(degp_ref)
  ssum = sp_ref[0, 0:_N, :] + sp_ref[1, 0:_N, :]
  h = jnp.maximum(dinv * (ssum + g_ref[...]) + b_ref[...], 0.0)
  out_ref[...] = dinv * jnp.dot(h, w_ref[...],
                                preferred_element_type=jnp.float32)


def _tc_final_body(sp_ref, g_ref, degp_ref, b_ref, out_ref):
  dinv = _dinv_of(degp_ref)
  ssum = sp_ref[0, 0:_N, :] + sp_ref[1, 0:_N, :]
  o = dinv * (ssum + g_ref[...]) + b_ref[...]
  m = jnp.max(o, axis=1, keepdims=True)
  lse = m + jnp.log(jnp.sum(jnp.exp(o - m), axis=1, keepdims=True))
  out_ref[...] = o - lse


_degree = _make_degree()
_scatter32 = _make_edge_scatter(32)
_scatter16 = _make_edge_scatter(16)

_tc_first = pl.pallas_call(
    _tc_first_body,
    out_shape=jax.ShapeDtypeStruct((_N, 32), jnp.float32))


def _tc_mid(width):
  return pl.pallas_call(
      _tc_mid_body,
      out_shape=jax.ShapeDtypeStruct((_N, width), jnp.float32))


_tc_final = pl.pallas_call(
    _tc_final_body,
    out_shape=jax.ShapeDtypeStruct((_N, 16), jnp.float32))


def kernel(x, edge_index, glove, W1, b1, W2, b2, W3, b3):
  pad = _EPAD - _E
  srcp = jnp.concatenate(
      [edge_index[0], jnp.zeros((pad,), jnp.int32)]).reshape(
          _NW, _NBLK, _BLKC * _CHUNK)
  dstp = jnp.concatenate(
      [edge_index[1], jnp.full((pad,), _N, jnp.int32)]).reshape(
          _NW, _NBLK, _BLKC * _CHUNK)
  ones = jnp.ones((_BLKC * _CHUNK, _DEGW), jnp.float32)
  z8 = jnp.zeros((_RPT, _DEGW), jnp.float32)
  z32 = jnp.zeros((_RPT, 32), jnp.float32)
  z16 = jnp.zeros((_RPT, 16), jnp.float32)

  degp = _degree(ones, dstp, z8)
  g1 = _tc_first(x, glove, W1, degp)
  s1 = _scatter32(g1, srcp, dstp, z32)
  g2 = _tc_mid(32)(s1, g1, degp, b1.reshape(1, -1), W2)
  s2 = _scatter32(g2, srcp, dstp, z32)
  g3 = _tc_mid(16)(s2, g2, degp, b2.reshape(1, -1), W3)
  s3 = _scatter16(g3, srcp, dstp, z16)
  return _tc_final(s3, g3, degp, b3.reshape(1, -1))


# speedup vs baseline: 1.2195x; 1.0665x over previous
import jax
import jax.numpy as jnp
from jax import lax
from jax.experimental import pallas as pl
from jax.experimental.pallas import tpu as pltpu
from jax.experimental.pallas import tpu_sc as plsc

_N = 10000
_E = 320000
_NSUB = 16
_NCORE = 2
_NW = _NSUB * _NCORE
_CHUNK = 128
_CHUNKS = 80
_BLKC = 8
_NBLK = _CHUNKS // _BLKC
_EPAD = _NW * _CHUNKS * _CHUNK
_NACC = 10112
_RPT = _NACC // _NSUB


def _make_edge_scatter(width):
  mesh = plsc.VectorSubcoreMesh(core_axis_name="c", subcore_axis_name="s")

  def body(table_hbm, src_hbm, dst_hbm, zeros_hbm, out_hbm,
           src_v, dst_v, rows0, rows1, sg0, sg1, acc_sh):
    c = lax.axis_index("c")
    s = lax.axis_index("s")
    wid = c * _NSUB + s
    pltpu.sync_copy(src_hbm.at[wid], src_v)
    pltpu.sync_copy(dst_hbm.at[wid], dst_v)
    sl = pl.ds(s * _RPT, _RPT)
    pltpu.sync_copy(zeros_hbm, acc_sh.at[sl])
    plsc.subcore_barrier()

    def gather(j, buf, sem):
      pltpu.async_copy(table_hbm.at[src_v.at[j]], buf, sem)

    def gather_wait(buf, sem):
      pltpu.make_async_copy(table_hbm.at[src_v.at[0]], buf, sem).wait()

    gather(0, rows0, sg0)

    def pair(t, carry):
      j0 = 2 * t
      gather(j0 + 1, rows1, sg1)
      gather_wait(rows0, sg0)
      pltpu.sync_copy(rows0, acc_sh.at[dst_v.at[j0]], add=True)

      @pl.when(t < _NBLK // 2 - 1)
      def _():
        gather(j0 + 2, rows0, sg0)

      gather_wait(rows1, sg1)
      pltpu.sync_copy(rows1, acc_sh.at[dst_v.at[j0 + 1]], add=True)
      return carry

    lax.fori_loop(0, _NBLK // 2, pair, 0)
    plsc.subcore_barrier()
    pltpu.sync_copy(acc_sh.at[sl], out_hbm.at[c, sl])

  return pl.kernel(
      body,
      out_type=jax.ShapeDtypeStruct((_NCORE, _NACC, width), jnp.float32),
      mesh=mesh,
      compiler_params=pltpu.CompilerParams(use_tc_tiling_on_sc=False),
      scratch_types=[
          pltpu.VMEM((_NBLK, _BLKC * _CHUNK), jnp.int32),
          pltpu.VMEM((_NBLK, _BLKC * _CHUNK), jnp.int32),
          pltpu.VMEM((_BLKC * _CHUNK, width), jnp.float32),
          pltpu.VMEM((_BLKC * _CHUNK, width), jnp.float32),
          pltpu.SemaphoreType.DMA,
          pltpu.SemaphoreType.DMA,
          pltpu.VMEM_SHARED((_NACC, width), jnp.float32),
      ],
  )


_DEGW = 8


def _make_degree():
  mesh = plsc.VectorSubcoreMesh(core_axis_name="c", subcore_axis_name="s")

  def body(ones_hbm, dst_hbm, zeros_hbm, out_hbm, dst_v, rows_v, acc_sh):
    c = lax.axis_index("c")
    s = lax.axis_index("s")
    wid = c * _NSUB + s
    pltpu.sync_copy(dst_hbm.at[wid], dst_v)
    pltpu.sync_copy(ones_hbm, rows_v)
    sl = pl.ds(s * _RPT, _RPT)
    pltpu.sync_copy(zeros_hbm, acc_sh.at[sl])
    plsc.subcore_barrier()

    def step(j, carry):
      pltpu.sync_copy(rows_v, acc_sh.at[dst_v.at[j]], add=True)
      return carry

    lax.fori_loop(0, _NBLK, step, 0)
    plsc.subcore_barrier()
    pltpu.sync_copy(acc_sh.at[sl], out_hbm.at[c, sl])

  return pl.kernel(
      body,
      out_type=jax.ShapeDtypeStruct((_NCORE, _NACC, _DEGW), jnp.float32),
      mesh=mesh,
      compiler_params=pltpu.CompilerParams(use_tc_tiling_on_sc=False),
      scratch_types=[
          pltpu.VMEM((_NBLK, _BLKC * _CHUNK), jnp.int32),
          pltpu.VMEM((_BLKC * _CHUNK, _DEGW), jnp.float32),
          pltpu.VMEM_SHARED((_NACC, _DEGW), jnp.float32),
      ],
  )


def _dinv_of(degp_ref):
  deg = degp_ref[0, 0:_N, 0:1] + degp_ref[1, 0:_N, 0:1] + 1.0
  return lax.rsqrt(deg)


def _tc_first_body(x_ref, glove_ref, w1_ref, degp_ref, g1_ref):
  dinv = _dinv_of(degp_ref)
  w1p = jnp.dot(glove_ref[...], w1_ref[...], preferred_element_type=jnp.float32)
  g1_ref[...] = dinv * jnp.dot(x_ref[...], w1p,
                               preferred_element_type=jnp.float32)


def _tc_mid_body(sp_ref, g_ref, degp_ref, b_ref, w_ref, out_ref):
  dinv = _dinv_of(degp_ref)
  ssum = sp_ref[0, 0:_N, :] + sp_ref[1, 0:_N, :]
  h = jnp.maximum(dinv * (ssum + g_ref[...]) + b_ref[...], 0.0)
  out_ref[...] = dinv * jnp.dot(h, w_ref[...],
                                preferred_element_type=jnp.float32)


def _tc_final_body(sp_ref, g_ref, degp_ref, b_ref, out_ref):
  dinv = _dinv_of(degp_ref)
  ssum = sp_ref[0, 0:_N, :] + sp_ref[1, 0:_N, :]
  o = dinv * (ssum + g_ref[...]) + b_ref[...]
  m = jnp.max(o, axis=1, keepdims=True)
  lse = m + jnp.log(jnp.sum(jnp.exp(o - m), axis=1, keepdims=True))
  out_ref[...] = o - lse


_degree = _make_degree()
_scatter32 = _make_edge_scatter(32)
_scatter16 = _make_edge_scatter(16)

_tc_first = pl.pallas_call(
    _tc_first_body,
    out_shape=jax.ShapeDtypeStruct((_N, 32), jnp.float32))


def _tc_mid(width):
  return pl.pallas_call(
      _tc_mid_body,
      out_shape=jax.ShapeDtypeStruct((_N, width), jnp.float32))


_tc_final = pl.pallas_call(
    _tc_final_body,
    out_shape=jax.ShapeDtypeStruct((_N, 16), jnp.float32))


def kernel(x, edge_index, glove, W1, b1, W2, b2, W3, b3):
  pad = _EPAD - _E
  srcp = jnp.concatenate(
      [edge_index[0], jnp.zeros((pad,), jnp.int32)]).reshape(
          _NW, _NBLK, _BLKC * _CHUNK)
  dstp = jnp.concatenate(
      [edge_index[1], jnp.full((pad,), _N, jnp.int32)]).reshape(
          _NW, _NBLK, _BLKC * _CHUNK)
  ones = jnp.ones((_BLKC * _CHUNK, _DEGW), jnp.float32)
  z8 = jnp.zeros((_RPT, _DEGW), jnp.float32)
  z32 = jnp.zeros((_RPT, 32), jnp.float32)
  z16 = jnp.zeros((_RPT, 16), jnp.float32)

  degp = _degree(ones, dstp, z8)
  g1 = _tc_first(x, glove, W1, degp)
  s1 = _scatter32(g1, srcp, dstp, z32)
  g2 = _tc_mid(32)(s1, g1, degp, b1.reshape(1, -1), W2)
  s2 = _scatter32(g2, srcp, dstp, z32)
  g3 = _tc_mid(16)(s2, g2, degp, b2.reshape(1, -1), W3)
  s3 = _scatter16(g3, srcp, dstp, z16)
  return _tc_final(s3, g3, degp, b3.reshape(1, -1))


# speedup vs baseline: 1.2540x; 1.0283x over previous
import jax
import jax.numpy as jnp
from jax import lax
from jax.experimental import pallas as pl
from jax.experimental.pallas import tpu as pltpu
from jax.experimental.pallas import tpu_sc as plsc

_N = 10000
_E = 320000
_NSUB = 16
_NCORE = 2
_NW = _NSUB * _NCORE
_CHUNK = 128
_CHUNKS = 80
_BLKC = 8
_NBLK = _CHUNKS // _BLKC
_EPAD = _NW * _CHUNKS * _CHUNK
_NACC = 10112
_RPT = _NACC // _NSUB


def _make_edge_scatter(width):
  mesh = plsc.VectorSubcoreMesh(core_axis_name="c", subcore_axis_name="s")

  def body(table_hbm, src_hbm, dst_hbm, zeros_hbm, out_hbm,
           src_v, dst_v, rows0, rows1, sg0, sg1, acc_sh):
    c = lax.axis_index("c")
    s = lax.axis_index("s")
    wid = c * _NSUB + s
    pltpu.sync_copy(src_hbm.at[wid], src_v)
    pltpu.sync_copy(dst_hbm.at[wid], dst_v)
    sl = pl.ds(s * _RPT, _RPT)
    pltpu.sync_copy(zeros_hbm, acc_sh.at[sl])
    plsc.subcore_barrier()

    def gather(j, buf, sem):
      pltpu.async_copy(table_hbm.at[src_v.at[j]], buf, sem)

    def gather_wait(buf, sem):
      pltpu.make_async_copy(table_hbm.at[src_v.at[0]], buf, sem).wait()

    gather(0, rows0, sg0)

    def pair(t, carry):
      j0 = 2 * t
      gather(j0 + 1, rows1, sg1)
      gather_wait(rows0, sg0)
      pltpu.sync_copy(rows0, acc_sh.at[dst_v.at[j0]], add=True)

      @pl.when(t < _NBLK // 2 - 1)
      def _():
        gather(j0 + 2, rows0, sg0)

      gather_wait(rows1, sg1)
      pltpu.sync_copy(rows1, acc_sh.at[dst_v.at[j0 + 1]], add=True)
      return carry

    lax.fori_loop(0, _NBLK // 2, pair, 0)
    plsc.subcore_barrier()
    pltpu.sync_copy(acc_sh.at[sl], out_hbm.at[c, sl])

  return pl.kernel(
      body,
      out_type=jax.ShapeDtypeStruct((_NCORE, _NACC, width), jnp.float32),
      mesh=mesh,
      compiler_params=pltpu.CompilerParams(use_tc_tiling_on_sc=False),
      scratch_types=[
          pltpu.VMEM((_NBLK, _BLKC * _CHUNK), jnp.int32),
          pltpu.VMEM((_NBLK, _BLKC * _CHUNK), jnp.int32),
          pltpu.VMEM((_BLKC * _CHUNK, width), jnp.float32),
          pltpu.VMEM((_BLKC * _CHUNK, width), jnp.float32),
          pltpu.SemaphoreType.DMA,
          pltpu.SemaphoreType.DMA,
          pltpu.VMEM_SHARED((_NACC, width), jnp.float32),
      ],
  )


_DEGW = 8


def _make_degree():
  mesh = plsc.VectorSubcoreMesh(core_axis_name="c", subcore_axis_name="s")

  def body(ones_hbm, dst_hbm, zeros_hbm, out_hbm, dst_v, rows_v, acc_sh):
    c = lax.axis_index("c")
    s = lax.axis_index("s")
    wid = c * _NSUB + s
    pltpu.sync_copy(dst_hbm.at[wid], dst_v)
    pltpu.sync_copy(ones_hbm, rows_v)
    sl = pl.ds(s * _RPT, _RPT)
    pltpu.sync_copy(zeros_hbm, acc_sh.at[sl])
    plsc.subcore_barrier()

    def step(j, carry):
      pltpu.sync_copy(rows_v, acc_sh.at[dst_v.at[j]], add=True)
      return carry

    lax.fori_loop(0, _NBLK, step, 0)
    plsc.subcore_barrier()
    pltpu.sync_copy(acc_sh.at[sl], out_hbm.at[c, sl])

  return pl.kernel(
      body,
      out_type=jax.ShapeDtypeStruct((_NCORE, _NACC, _DEGW), jnp.float32),
      mesh=mesh,
      compiler_params=pltpu.CompilerParams(use_tc_tiling_on_sc=False),
      scratch_types=[
          pltpu.VMEM((_NBLK, _BLKC * _CHUNK), jnp.int32),
          pltpu.VMEM((_BLKC * _CHUNK, _DEGW), jnp.float32),
          pltpu.VMEM_SHARED((_NACC, _DEGW), jnp.float32),
      ],
  )


def _dinv_of(degp_ref):
  deg = degp_ref[0, 0:_N, 0:1] + degp_ref[1, 0:_N, 0:1] + 1.0
  return lax.rsqrt(deg)


def _tc_first_body(x_ref, glove_ref, w1_ref, degp_ref, g1_ref):
  dinv = _dinv_of(degp_ref)
  w1p = jnp.dot(glove_ref[...], w1_ref[...], preferred_element_type=jnp.float32)
  g1_ref[...] = dinv * jnp.dot(x_ref[...], w1p,
                               preferred_element_type=jnp.float32)


def _tc_mid_body(sp_ref, g_ref, degp_ref, b_ref, w_ref, out_ref):
  dinv = _dinv_of(degp_ref)
  ssum = sp_ref[0, 0:_N, :] + sp_ref[1, 0:_N, :]
  h = jnp.maximum(dinv * (ssum + g_ref[...]) + b_ref[...], 0.0)
  out_ref[...] = dinv * jnp.dot(h, w_ref[...],
                                preferred_element_type=jnp.float32)


def _tc_final_body(sp_ref, g_ref, degp_ref, b_ref, out_ref):
  dinv = _dinv_of(degp_ref)
  ssum = sp_ref[0, 0:_N, :] + sp_ref[1, 0:_N, :]
  o = dinv * (ssum + g_ref[...]) + b_ref[...]
  m = jnp.max(o, axis=1, keepdims=True)
  lse = m + jnp.log(jnp.sum(jnp.exp(o - m), axis=1, keepdims=True))
  out_ref[...] = o - lse


_degree = _make_degree()
_scatter32 = _make_edge_scatter(32)
_scatter16 = _make_edge_scatter(16)

_tc_first = pl.pallas_call(
    _tc_first_body,
    out_shape=jax.ShapeDtypeStruct((_N, 32), jnp.float32))


def _tc_mid(width):
  return pl.pallas_call(
      _tc_mid_body,
      out_shape=jax.ShapeDtypeStruct((_N, width), jnp.float32))


_tc_final = pl.pallas_call(
    _tc_final_body,
    out_shape=jax.ShapeDtypeStruct((_N, 16), jnp.float32))


def kernel(x, edge_index, glove, W1, b1, W2, b2, W3, b3):
  pad = _EPAD - _E
  srcp = jnp.concatenate(
      [edge_index[0], jnp.zeros((pad,), jnp.int32)]).reshape(
          _NW, _NBLK, _BLKC * _CHUNK)
  sink = _N + jnp.arange(pad, dtype=jnp.int32) % (_NACC - _N)
  dstp = jnp.concatenate([edge_index[1], sink]).reshape(
      _NW, _NBLK, _BLKC * _CHUNK)
  ones = jnp.ones((_BLKC * _CHUNK, _DEGW), jnp.float32)
  z8 = jnp.zeros((_RPT, _DEGW), jnp.float32)
  z32 = jnp.zeros((_RPT, 32), jnp.float32)
  z16 = jnp.zeros((_RPT, 16), jnp.float32)

  degp = _degree(ones, dstp, z8)
  g1 = _tc_first(x, glove, W1, degp)
  s1 = _scatter32(g1, srcp, dstp, z32)
  g2 = _tc_mid(32)(s1, g1, degp, b1.reshape(1, -1), W2)
  s2 = _scatter32(g2, srcp, dstp, z32)
  g3 = _tc_mid(16)(s2, g2, degp, b2.reshape(1, -1), W3)
  s3 = _scatter16(g3, srcp, dstp, z16)
  return _tc_final(s3, g3, degp, b3.reshape(1, -1))
